# baseline (speedup 1.0000x reference)
import functools

import jax
import jax.numpy as jnp
from jax.experimental import pallas as pl
from jax.experimental.pallas import tpu as pltpu

_NUM_CLASSES = 10
_BN_EPS = 1e-5
_LANE = 128
_VMEM_LIMIT = 32 * 1024 * 1024


def _fold_bn(conv_b, gamma, beta, run_mean, run_var):
    inv_std = 1.0 / jnp.sqrt(run_var + _BN_EPS)
    scale = gamma * inv_std
    shift = (conv_b - run_mean) * scale + beta
    return (scale.reshape(1, -1).astype(jnp.float32),
            shift.reshape(1, -1).astype(jnp.float32))


def _tap_major(conv_w):
    cout, cin = conv_w.shape[0], conv_w.shape[1]
    w = jnp.transpose(conv_w, (2, 3, 1, 0)).reshape(9, cin, cout)
    return w.astype(jnp.bfloat16)


def _bn_relu_pool(acc, scale, shift, h, w):
    c = acc.shape[-1]
    y = jnp.maximum(acc * scale + shift, 0.0)
    y = jnp.max(y.reshape(h * (w // 2), 2, c), axis=1)
    y = jnp.max(y.reshape(h // 2, 2, w // 2, c), axis=1)
    return y


def _fused_convs_kernel(xp_ref, w1_ref, s1_ref, t1_ref,
                        w2_ref, s2_ref, t2_ref,
                        w3_ref, s3_ref, t3_ref, o_ref, *, H, W):
    h2, w2 = H // 2, W // 2
    dn = (((0,), (0,)), ((), ()))
    acc_e = jax.lax.dot_general(xp_ref[0], w1_ref[...], dimension_numbers=dn,
                                preferred_element_type=jnp.float32)
    acc_o = jax.lax.dot_general(xp_ref[1], w1_ref[...], dimension_numbers=dn,
                                preferred_element_type=jnp.float32)
    s1, t1 = s1_ref[...], t1_ref[...]
    ye = jnp.maximum(acc_e * s1 + t1, 0.0)
    yo = jnp.maximum(acc_o * s1 + t1, 0.0)
    y = jnp.maximum(ye, yo)
    y1 = jnp.max(y.reshape(H // 2, 2, w2, 64), axis=1)

    y1p = jnp.pad(y1.astype(jnp.bfloat16), ((1, 1), (1, 1), (0, 0)))
    acc2 = jnp.zeros((h2 * w2, 128), jnp.float32)
    for dy in range(3):
        for dx in range(3):
            a2 = y1p[dy:dy + h2, dx:dx + w2, :].reshape(h2 * w2, 64)
            acc2 = acc2 + jnp.dot(a2, w2_ref[dy * 3 + dx],
                                  preferred_element_type=jnp.float32)
    y2 = _bn_relu_pool(acc2, s2_ref[...], t2_ref[...], h2, w2)
    h3, w3 = h2 // 2, w2 // 2

    y2p = jnp.pad(y2.astype(jnp.bfloat16), ((1, 1), (1, 1), (0, 0)))
    acc3 = jnp.zeros((h3 * w3, 512), jnp.float32)
    for dy in range(3):
        for dx in range(3):
            a3 = y2p[dy:dy + h3, dx:dx + w3, :].reshape(h3 * w3, 128)
            acc3 = acc3 + jnp.dot(a3, w3_ref[dy * 3 + dx],
                                  preferred_element_type=jnp.float32)
    y3 = _bn_relu_pool(acc3, s3_ref[...], t3_ref[...], h3, w3)

    hw = (h3 // 2) * (w3 // 2)
    g = y3.astype(jnp.bfloat16).reshape(hw, 512).astype(jnp.float32)
    o_ref[...] = jnp.mean(g, axis=0, keepdims=True)


def _head_kernel(g_ref, w1_ref, b1_ref, w2_ref, b2_ref, o_ref):
    h = jnp.dot(g_ref[...].astype(jnp.bfloat16), w1_ref[...],
                preferred_element_type=jnp.float32) + b1_ref[...]
    h = jnp.maximum(h, 0.0)
    out = jnp.dot(h.astype(jnp.bfloat16), w2_ref[...],
                  preferred_element_type=jnp.float32) + b2_ref[...]
    o_ref[...] = out


def _pack_conv1_input(x_nchw):
    n, _, h, w = x_nchw.shape
    xb = x_nchw.astype(jnp.bfloat16)
    xp = jnp.pad(xb, ((0, 0), (0, 0), (1, 1), (1, 1)))
    xsp = [xp[:, :, :, 0::2], xp[:, :, :, 1::2]]
    halves = []
    for b0 in range(2):
        taps = [xsp[(b0 + dx) % 2][:, :, dy:dy + h,
                                   (b0 + dx) // 2:(b0 + dx) // 2 + w // 2]
                for dy in range(3) for dx in range(3)]
        halves.append(jnp.stack(taps, axis=1).reshape(n, 27, h * (w // 2)))
    pk = jnp.stack(halves, axis=1)
    return jnp.pad(pk, ((0, 0), (0, 0), (0, 5), (0, 0)))


def kernel(c1_w, c1_b, c1_gamma, c1_beta, c1_mean, c1_var,
           c2_w, c2_b, c2_gamma, c2_beta, c2_mean, c2_var,
           c3_w, c3_b, c3_gamma, c3_beta, c3_mean, c3_var,
           fc1_w, fc1_b, fc2_w, fc2_b, x_nchw):
    n, _, h, w = x_nchw.shape

    xpk = _pack_conv1_input(x_nchw)

    w1 = jnp.transpose(c1_w, (2, 3, 1, 0)).reshape(27, 64)
    w1 = jnp.pad(w1, ((0, 5), (0, 0))).astype(jnp.bfloat16)
    s1, t1 = _fold_bn(c1_b, c1_gamma, c1_beta, c1_mean, c1_var)
    w2m = _tap_major(c2_w)
    s2, t2 = _fold_bn(c2_b, c2_gamma, c2_beta, c2_mean, c2_var)
    w3m = _tap_major(c3_w)
    s3, t3 = _fold_bn(c3_b, c3_gamma, c3_beta, c3_mean, c3_var)

    body = functools.partial(_fused_convs_kernel, H=h, W=w)
    gap = pl.pallas_call(
        body,
        out_shape=jax.ShapeDtypeStruct((n, 1, 512), jnp.float32),
        grid=(n,),
        in_specs=[
            pl.BlockSpec((None, 2, 32, h * w // 2), lambda i: (i, 0, 0, 0)),
            pl.BlockSpec((32, 64), lambda i: (0, 0)),
            pl.BlockSpec((1, 64), lambda i: (0, 0)),
            pl.BlockSpec((1, 64), lambda i: (0, 0)),
            pl.BlockSpec((9, 64, 128), lambda i: (0, 0, 0)),
            pl.BlockSpec((1, 128), lambda i: (0, 0)),
            pl.BlockSpec((1, 128), lambda i: (0, 0)),
            pl.BlockSpec((9, 128, 512), lambda i: (0, 0, 0)),
            pl.BlockSpec((1, 512), lambda i: (0, 0)),
            pl.BlockSpec((1, 512), lambda i: (0, 0)),
        ],
        out_specs=pl.BlockSpec((None, 1, 512), lambda i: (i, 0, 0)),
        compiler_params=pltpu.CompilerParams(
            dimension_semantics=("parallel",),
            vmem_limit_bytes=_VMEM_LIMIT,
        ),
    )(xpk, w1, s1, t1, w2m, s2, t2, w3m, s3, t3)
    g = gap.reshape(n, 512)

    w1f = fc1_w.astype(jnp.bfloat16)
    b1f = fc1_b.reshape(1, -1).astype(jnp.float32)
    npad = _LANE
    w2f = jnp.pad(fc2_w, ((0, 0), (0, npad - _NUM_CLASSES))).astype(jnp.bfloat16)
    b2f = jnp.pad(fc2_b, (0, npad - _NUM_CLASSES)).reshape(1, -1).astype(jnp.float32)

    logits = pl.pallas_call(
        _head_kernel,
        out_shape=jax.ShapeDtypeStruct((n, npad), jnp.float32),
        grid=(1,),
        in_specs=[
            pl.BlockSpec((n, 512), lambda i: (0, 0)),
            pl.BlockSpec((512, 1024), lambda i: (0, 0)),
            pl.BlockSpec((1, 1024), lambda i: (0, 0)),
            pl.BlockSpec((1024, npad), lambda i: (0, 0)),
            pl.BlockSpec((1, npad), lambda i: (0, 0)),
        ],
        out_specs=pl.BlockSpec((n, npad), lambda i: (0, 0)),
        compiler_params=pltpu.CompilerParams(
            dimension_semantics=("arbitrary",),
            vmem_limit_bytes=_VMEM_LIMIT,
        ),
    )(g, w1f, b1f, w2f, b2f)
    return logits[:, :_NUM_CLASSES]


# speedup vs baseline: 1.5631x; 1.5631x over previous
import functools

import jax
import jax.numpy as jnp
from jax.experimental import pallas as pl
from jax.experimental.pallas import tpu as pltpu

_NUM_CLASSES = 10
_BN_EPS = 1e-5
_LANE = 128
_VMEM_LIMIT = 32 * 1024 * 1024


def _fold_bn(conv_b, gamma, beta, run_mean, run_var):
    inv_std = 1.0 / jnp.sqrt(run_var + _BN_EPS)
    scale = gamma * inv_std
    shift = (conv_b - run_mean) * scale + beta
    return (scale.reshape(1, -1).astype(jnp.float32),
            shift.reshape(1, -1).astype(jnp.float32))


def _tap_major(conv_w):
    cout, cin = conv_w.shape[0], conv_w.shape[1]
    w = jnp.transpose(conv_w, (2, 3, 1, 0)).reshape(9, cin, cout)
    return w.astype(jnp.bfloat16)


def _bn_relu_pool(acc, scale, shift, h, w):
    c = acc.shape[-1]
    y = jnp.maximum(acc * scale + shift, 0.0)
    y = jnp.max(y.reshape(h * (w // 2), 2, c), axis=1)
    y = jnp.max(y.reshape(h // 2, 2, w // 2, c), axis=1)
    return y


def _fused_convs_kernel(xp_ref, w1_ref, s1_ref, t1_ref,
                        w2_ref, s2_ref, t2_ref,
                        w3_ref, s3_ref, t3_ref, o_ref, *, H, W):
    dn = (((0,), (0,)), ((), ()))

    acc_e = jax.lax.dot_general(xp_ref[0], w1_ref[...], dimension_numbers=dn,
                                preferred_element_type=jnp.float32)
    acc_o = jax.lax.dot_general(xp_ref[1], w1_ref[...], dimension_numbers=dn,
                                preferred_element_type=jnp.float32)
    s1, t1 = s1_ref[...], t1_ref[...]
    y = jnp.maximum(jnp.maximum(acc_e * s1 + t1, 0.0),
                    jnp.maximum(acc_o * s1 + t1, 0.0))
    y = jnp.max(y.reshape(2, 2, H // 2, 2, 8, 64), axis=3)
    y1p = jnp.pad(y.astype(jnp.bfloat16),
                  ((0, 0), (0, 0), (1, 1), (1, 1), (0, 0)))

    h2 = H // 2
    accs2 = []
    for g2 in range(2):
        acc = jnp.zeros((h2 * 16, 128), jnp.float32)
        for dy in range(3):
            for dx in range(3):
                e = g2 + dx - 1
                eta, eps = e % 2, (e - e % 2) // 2
                pieces = []
                for s3 in range(2):
                    lam = (s3 + eps) % 2
                    kap = (s3 + eps - lam) // 2
                    pieces.append(y1p[eta, lam, dy:dy + h2,
                                      kap + 1:kap + 9, :])
                a = jnp.stack(pieces, axis=0).reshape(h2 * 16, 64)
                acc = acc + jnp.dot(a, w2_ref[dy * 3 + dx],
                                    preferred_element_type=jnp.float32)
        accs2.append(acc)
    s2, t2 = s2_ref[...], t2_ref[...]
    z = jnp.maximum(jnp.maximum(accs2[0] * s2 + t2, 0.0),
                    jnp.maximum(accs2[1] * s2 + t2, 0.0))
    z = jnp.max(z.reshape(2, h2 // 2, 2, 8, 128), axis=2)
    y2p = jnp.pad(z.astype(jnp.bfloat16),
                  ((0, 0), (1, 1), (1, 1), (0, 0)))

    h3 = H // 4
    accs3 = []
    for g4 in range(2):
        acc = jnp.zeros((h3 * 8, 512), jnp.float32)
        for dy in range(3):
            for dx in range(3):
                e = g4 + dx - 1
                eta, eps = e % 2, (e - e % 2) // 2
                a = y2p[eta, dy:dy + h3, eps + 1:eps + 9, :].reshape(h3 * 8, 128)
                acc = acc + jnp.dot(a, w3_ref[dy * 3 + dx],
                                    preferred_element_type=jnp.float32)
        accs3.append(acc)
    s3_, t3_ = s3_ref[...], t3_ref[...]
    u = jnp.maximum(jnp.maximum(accs3[0] * s3_ + t3_, 0.0),
                    jnp.maximum(accs3[1] * s3_ + t3_, 0.0))
    u = jnp.max(u.reshape(h3 // 2, 2, 8, 512), axis=1)

    g = u.astype(jnp.bfloat16).reshape((h3 // 2) * 8, 512).astype(jnp.float32)
    o_ref[...] = jnp.mean(g, axis=0, keepdims=True)


def _head_kernel(g_ref, w1_ref, b1_ref, w2_ref, b2_ref, o_ref):
    h = jnp.dot(g_ref[...].astype(jnp.bfloat16), w1_ref[...],
                preferred_element_type=jnp.float32) + b1_ref[...]
    h = jnp.maximum(h, 0.0)
    out = jnp.dot(h.astype(jnp.bfloat16), w2_ref[...],
                  preferred_element_type=jnp.float32) + b2_ref[...]
    o_ref[...] = out


def _pack_conv1_input(x_nchw):
    n, _, h, w = x_nchw.shape
    xb = x_nchw.astype(jnp.bfloat16)
    xp = jnp.pad(xb, ((0, 0), (0, 0), (1, 1), (1, 1)))
    taps = [xp[:, :, dy:dy + h, dx:dx + w]
            for dy in range(3) for dx in range(3)]
    pk = jnp.stack(taps, axis=1)
    pk = pk.reshape(n, 27, h, w // 8, 2, 2, 2)
    pk = jnp.transpose(pk, (0, 6, 1, 5, 4, 2, 3))
    pk = pk.reshape(n, 2, 27, h * (w // 2))
    return jnp.pad(pk, ((0, 0), (0, 0), (0, 5), (0, 0)))


def kernel(c1_w, c1_b, c1_gamma, c1_beta, c1_mean, c1_var,
           c2_w, c2_b, c2_gamma, c2_beta, c2_mean, c2_var,
           c3_w, c3_b, c3_gamma, c3_beta, c3_mean, c3_var,
           fc1_w, fc1_b, fc2_w, fc2_b, x_nchw):
    n, _, h, w = x_nchw.shape

    xpk = _pack_conv1_input(x_nchw)

    w1 = jnp.transpose(c1_w, (2, 3, 1, 0)).reshape(27, 64)
    w1 = jnp.pad(w1, ((0, 5), (0, 0))).astype(jnp.bfloat16)
    s1, t1 = _fold_bn(c1_b, c1_gamma, c1_beta, c1_mean, c1_var)
    w2m = _tap_major(c2_w)
    s2, t2 = _fold_bn(c2_b, c2_gamma, c2_beta, c2_mean, c2_var)
    w3m = _tap_major(c3_w)
    s3, t3 = _fold_bn(c3_b, c3_gamma, c3_beta, c3_mean, c3_var)

    body = functools.partial(_fused_convs_kernel, H=h, W=w)
    gap = pl.pallas_call(
        body,
        out_shape=jax.ShapeDtypeStruct((n, 1, 512), jnp.float32),
        grid=(n,),
        in_specs=[
            pl.BlockSpec((None, 2, 32, h * w // 2), lambda i: (i, 0, 0, 0)),
            pl.BlockSpec((32, 64), lambda i: (0, 0)),
            pl.BlockSpec((1, 64), lambda i: (0, 0)),
            pl.BlockSpec((1, 64), lambda i: (0, 0)),
            pl.BlockSpec((9, 64, 128), lambda i: (0, 0, 0)),
            pl.BlockSpec((1, 128), lambda i: (0, 0)),
            pl.BlockSpec((1, 128), lambda i: (0, 0)),
            pl.BlockSpec((9, 128, 512), lambda i: (0, 0, 0)),
            pl.BlockSpec((1, 512), lambda i: (0, 0)),
            pl.BlockSpec((1, 512), lambda i: (0, 0)),
        ],
        out_specs=pl.BlockSpec((None, 1, 512), lambda i: (i, 0, 0)),
        compiler_params=pltpu.CompilerParams(
            dimension_semantics=("arbitrary",),
            vmem_limit_bytes=_VMEM_LIMIT,
        ),
    )(xpk, w1, s1, t1, w2m, s2, t2, w3m, s3, t3)
    g = gap.reshape(n, 512)

    w1f = fc1_w.astype(jnp.bfloat16)
    b1f = fc1_b.reshape(1, -1).astype(jnp.float32)
    npad = _LANE
    w2f = jnp.pad(fc2_w, ((0, 0), (0, npad - _NUM_CLASSES))).astype(jnp.bfloat16)
    b2f = jnp.pad(fc2_b, (0, npad - _NUM_CLASSES)).reshape(1, -1).astype(jnp.float32)

    logits = pl.pallas_call(
        _head_kernel,
        out_shape=jax.ShapeDtypeStruct((n, npad), jnp.float32),
        grid=(1,),
        in_specs=[
            pl.BlockSpec((n, 512), lambda i: (0, 0)),
            pl.BlockSpec((512, 1024), lambda i: (0, 0)),
            pl.BlockSpec((1, 1024), lambda i: (0, 0)),
            pl.BlockSpec((1024, npad), lambda i: (0, 0)),
            pl.BlockSpec((1, npad), lambda i: (0, 0)),
        ],
        out_specs=pl.BlockSpec((n, npad), lambda i: (0, 0)),
        compiler_params=pltpu.CompilerParams(
            dimension_semantics=("arbitrary",),
            vmem_limit_bytes=_VMEM_LIMIT,
        ),
    )(g, w1f, b1f, w2f, b2f)
    return logits[:, :_NUM_CLASSES]
